# baseline (device time: 81428 ns/iter reference)
import jax
import jax.numpy as jnp
from jax import lax
from jax.experimental import pallas as pl
from jax.experimental.pallas import tpu as pltpu

N_DEV = 16
SQ = 1024
HQ = 8
DH = 128
D = HQ * DH
BLK = 64
N_QB = SQ // BLK
N_PHASE = 4
SKV_SHARD = 1024
SCALE = 0.08838834764831843


def kernel(x, Wq, K_ext, V_ext, Wo):
    def body(x_ref, wq_ref, k_ref, v_ref, wo_ref, out_ref,
             send_buf, recv_buf, out_comm, send_sems1, recv_sems1,
             send_sems2, recv_sems2):
        my = lax.axis_index("i")

        xb = x_ref[0].astype(jnp.bfloat16)
        wqb = wq_ref[...].astype(jnp.bfloat16)
        kb16 = k_ref[0].reshape(SKV_SHARD, D).astype(jnp.bfloat16)
        vb16 = v_ref[0].reshape(SKV_SHARD, D).astype(jnp.bfloat16)

        def r1_rdma(e):
            return pltpu.make_async_remote_copy(
                src_ref=send_buf.at[e],
                dst_ref=recv_buf.at[my],
                send_sem=send_sems1.at[e],
                recv_sem=recv_sems1.at[my],
                device_id=(e,),
                device_id_type=pl.DeviceIdType.MESH,
            )

        for p in range(N_PHASE):
            blocks = [p + N_PHASE * a for a in range(N_QB // N_PHASE)]
            xp = jnp.concatenate(
                [xb[b * BLK:(b + 1) * BLK] for b in blocks], axis=0)
            qp = jnp.dot(xp, wqb, preferred_element_type=jnp.float32
                         ).astype(jnp.bfloat16)
            kp = jnp.concatenate(
                [kb16[b * BLK:(b + 1) * BLK] for b in blocks], axis=0)
            vp = jnp.concatenate(
                [vb16[b * BLK:(b + 1) * BLK] for b in blocks], axis=0)
            ctx_h = []
            l_h = []
            for h in range(HQ):
                qh = qp[:, h * DH:(h + 1) * DH]
                kh = kp[:, h * DH:(h + 1) * DH]
                s = lax.dot_general(
                    qh, kh, (((1,), (1,)), ((), ())),
                    preferred_element_type=jnp.float32) * SCALE
                w = jnp.exp(s)
                l_h.append(jnp.sum(w, axis=1, keepdims=True))
                ctx_h.append(jnp.dot(
                    w.astype(jnp.bfloat16), vp[:, h * DH:(h + 1) * DH],
                    preferred_element_type=jnp.float32))
            ctx_p = jnp.concatenate(ctx_h, axis=1)
            l_p = jnp.concatenate(l_h, axis=1)

            for a in range(N_QB // N_PHASE):
                e = p + N_PHASE * a
                send_buf[e, :, :D] = (
                    ctx_p[a * BLK:(a + 1) * BLK].astype(jnp.bfloat16))
                send_buf[e, :, D:] = (
                    l_p[a * BLK:(a + 1) * BLK].astype(jnp.bfloat16))

                @pl.when(e != my)
                def _():
                    r1_rdma(e).start()

        recv_buf[pl.ds(my, 1)] = send_buf[pl.ds(my, 1)]

        for o in range(1, N_DEV):
            s = (my - o) % N_DEV
            rdma = pltpu.make_async_remote_copy(
                src_ref=send_buf.at[s],
                dst_ref=recv_buf.at[s],
                send_sem=send_sems1.at[s],
                recv_sem=recv_sems1.at[s],
                device_id=(s,),
                device_id_type=pl.DeviceIdType.MESH,
            )
            rdma.wait_recv()

        acc = recv_buf[0].astype(jnp.float32)
        for src in range(1, N_DEV):
            acc = acc + recv_buf[src].astype(jnp.float32)
        ctx_sum = acc[:, :D]
        l_sum = acc[:, D:]
        attn = jnp.concatenate(
            [ctx_sum[:, h * DH:(h + 1) * DH] / l_sum[:, h:h + 1]
             for h in range(HQ)], axis=1)
        y = jnp.dot(attn.astype(jnp.bfloat16),
                    wo_ref[...].astype(jnp.bfloat16),
                    preferred_element_type=jnp.float32)
        out_ref[0, pl.ds(my * BLK, BLK), :] = y
        out_comm[pl.ds(my * BLK, BLK), :] = y.astype(jnp.bfloat16)

        r2 = []
        for o in range(1, N_DEV):
            e = (my + o) % N_DEV
            rdma = pltpu.make_async_remote_copy(
                src_ref=out_comm.at[pl.ds(my * BLK, BLK), :],
                dst_ref=out_comm.at[pl.ds(my * BLK, BLK), :],
                send_sem=send_sems2.at[o],
                recv_sem=recv_sems2.at[my],
                device_id=(e,),
                device_id_type=pl.DeviceIdType.MESH,
            )
            rdma.start()
            r2.append(rdma)
        for e in range(N_DEV):
            @pl.when(e != my)
            def _():
                r1_rdma(e).wait_send()
        for o in range(1, N_DEV):
            s = (my - o) % N_DEV
            rdma = pltpu.make_async_remote_copy(
                src_ref=out_comm.at[pl.ds(s * BLK, BLK), :],
                dst_ref=out_comm.at[pl.ds(s * BLK, BLK), :],
                send_sem=send_sems2.at[o],
                recv_sem=recv_sems2.at[s],
                device_id=(s,),
                device_id_type=pl.DeviceIdType.MESH,
            )
            rdma.wait_recv()
            out_ref[0, pl.ds(s * BLK, BLK), :] = (
                out_comm[pl.ds(s * BLK, BLK), :].astype(jnp.float32))
        for r in r2:
            r.wait_send()

    return pl.pallas_call(
        body,
        out_shape=jax.ShapeDtypeStruct((1, SQ, SQ), jnp.float32),
        in_specs=[pl.BlockSpec(memory_space=pltpu.VMEM)] * 5,
        out_specs=pl.BlockSpec(memory_space=pltpu.VMEM),
        scratch_shapes=[
            pltpu.VMEM((N_DEV, BLK, D + HQ), jnp.bfloat16),
            pltpu.VMEM((N_DEV, BLK, D + HQ), jnp.bfloat16),
            pltpu.VMEM((SQ, D), jnp.bfloat16),
            pltpu.SemaphoreType.DMA((N_DEV,)),
            pltpu.SemaphoreType.DMA((N_DEV,)),
            pltpu.SemaphoreType.DMA((N_DEV,)),
            pltpu.SemaphoreType.DMA((N_DEV,)),
        ],
    )(x, Wq, K_ext, V_ext, Wo)


# device time: 61820 ns/iter; 1.3172x vs baseline; 1.3172x over previous
import jax
import jax.numpy as jnp
from jax import lax
from jax.experimental import pallas as pl
from jax.experimental.pallas import tpu as pltpu

N_DEV = 16
SQ = 1024
HQ = 8
DH = 128
D = HQ * DH
BLK = 64
N_QB = SQ // BLK
N_PHASE = 4
SKV_SHARD = 1024
SCALE = 0.08838834764831843


def kernel(x, Wq, K_ext, V_ext, Wo):
    def body(x_ref, wq_ref, k_ref, v_ref, wo_ref, out_ref,
             send_buf, recv_buf, l_buf, recv_l, out_comm,
             send_sems1, recv_sems1, send_sems2, recv_sems2,
             send_sems3, recv_sems3):
        my = lax.axis_index("i")

        xb = x_ref[0].astype(jnp.bfloat16)
        wqb = wq_ref[...].astype(jnp.bfloat16)
        q = jnp.dot(xb, wqb, preferred_element_type=jnp.float32)
        qb16 = q.astype(jnp.bfloat16)
        kb16 = k_ref[0].reshape(SKV_SHARD, D).astype(jnp.bfloat16)
        vb16 = v_ref[0].reshape(SKV_SHARD, D).astype(jnp.bfloat16)

        for p in range(N_PHASE):
            blocks = [p + N_PHASE * a for a in range(N_QB // N_PHASE)]
            qp = jnp.concatenate(
                [qb16[b * BLK:(b + 1) * BLK] for b in blocks], axis=0)
            kp = jnp.concatenate(
                [kb16[b * BLK:(b + 1) * BLK] for b in blocks], axis=0)
            vp = jnp.concatenate(
                [vb16[b * BLK:(b + 1) * BLK] for b in blocks], axis=0)
            ctx_h = []
            l_h = []
            for h in range(HQ):
                qh = qp[:, h * DH:(h + 1) * DH]
                kh = kp[:, h * DH:(h + 1) * DH]
                s = lax.dot_general(
                    qh, kh, (((1,), (1,)), ((), ())),
                    preferred_element_type=jnp.float32) * SCALE
                w = jnp.exp(s)
                l_h.append(jnp.sum(w, axis=1, keepdims=True))
                ctx_h.append(jnp.dot(
                    w.astype(jnp.bfloat16), vp[:, h * DH:(h + 1) * DH],
                    preferred_element_type=jnp.float32))
            ctx_p = jnp.concatenate(ctx_h, axis=1)
            l_p = jnp.concatenate(l_h, axis=1)

            for a in range(N_QB // N_PHASE):
                e = p + N_PHASE * a
                send_buf[e] = (
                    ctx_p[a * BLK:(a + 1) * BLK].astype(jnp.float8_e4m3fn))
                l_buf[pl.ds(e * BLK, BLK)] = (
                    l_p[a * BLK:(a + 1) * BLK].astype(jnp.bfloat16))

        recv_buf[pl.ds(my, 1)] = send_buf[pl.ds(my, 1)]
        recv_l[pl.ds(my, 1)] = (
            l_buf[pl.ds(my * BLK, BLK)].reshape(1, BLK, HQ))

        r1 = []
        for o in range(1, N_DEV):
            e = (my + o) % N_DEV
            rdma = pltpu.make_async_remote_copy(
                src_ref=send_buf.at[e],
                dst_ref=recv_buf.at[my],
                send_sem=send_sems1.at[o],
                recv_sem=recv_sems1.at[my],
                device_id=(e,),
                device_id_type=pl.DeviceIdType.MESH,
            )
            rdma.start()
            r1.append(rdma)
            rdma_l = pltpu.make_async_remote_copy(
                src_ref=l_buf.at[pl.ds(e * BLK, BLK)],
                dst_ref=recv_l.at[my],
                send_sem=send_sems3.at[o],
                recv_sem=recv_sems3.at[my],
                device_id=(e,),
                device_id_type=pl.DeviceIdType.MESH,
            )
            rdma_l.start()
            r1.append(rdma_l)
        for o in range(1, N_DEV):
            s = (my - o) % N_DEV
            rdma = pltpu.make_async_remote_copy(
                src_ref=send_buf.at[s],
                dst_ref=recv_buf.at[s],
                send_sem=send_sems1.at[o],
                recv_sem=recv_sems1.at[s],
                device_id=(s,),
                device_id_type=pl.DeviceIdType.MESH,
            )
            rdma.wait_recv()
            rdma_l = pltpu.make_async_remote_copy(
                src_ref=l_buf.at[pl.ds(s * BLK, BLK)],
                dst_ref=recv_l.at[s],
                send_sem=send_sems3.at[o],
                recv_sem=recv_sems3.at[s],
                device_id=(s,),
                device_id_type=pl.DeviceIdType.MESH,
            )
            rdma_l.wait_recv()

        ctx_sum = recv_buf[0].astype(jnp.float32)
        l_sum = recv_l[0].astype(jnp.float32)
        for src in range(1, N_DEV):
            ctx_sum = ctx_sum + recv_buf[src].astype(jnp.float32)
            l_sum = l_sum + recv_l[src].astype(jnp.float32)
        attn = jnp.concatenate(
            [ctx_sum[:, h * DH:(h + 1) * DH] / l_sum[:, h:h + 1]
             for h in range(HQ)], axis=1)
        y = jnp.dot(attn.astype(jnp.bfloat16),
                    wo_ref[...].astype(jnp.bfloat16),
                    preferred_element_type=jnp.float32)
        out_ref[0, pl.ds(my * BLK, BLK), :] = y
        out_comm[pl.ds(my * BLK, BLK), :] = y.astype(jnp.bfloat16)

        r2 = []
        for o in range(1, N_DEV):
            e = (my + o) % N_DEV
            rdma = pltpu.make_async_remote_copy(
                src_ref=out_comm.at[pl.ds(my * BLK, BLK), :],
                dst_ref=out_comm.at[pl.ds(my * BLK, BLK), :],
                send_sem=send_sems2.at[o],
                recv_sem=recv_sems2.at[my],
                device_id=(e,),
                device_id_type=pl.DeviceIdType.MESH,
            )
            rdma.start()
            r2.append(rdma)
        for r in r1:
            r.wait_send()
        for o in range(1, N_DEV):
            s = (my - o) % N_DEV
            rdma = pltpu.make_async_remote_copy(
                src_ref=out_comm.at[pl.ds(s * BLK, BLK), :],
                dst_ref=out_comm.at[pl.ds(s * BLK, BLK), :],
                send_sem=send_sems2.at[o],
                recv_sem=recv_sems2.at[s],
                device_id=(s,),
                device_id_type=pl.DeviceIdType.MESH,
            )
            rdma.wait_recv()
            out_ref[0, pl.ds(s * BLK, BLK), :] = (
                out_comm[pl.ds(s * BLK, BLK), :].astype(jnp.float32))
        for r in r2:
            r.wait_send()

    return pl.pallas_call(
        body,
        out_shape=jax.ShapeDtypeStruct((1, SQ, SQ), jnp.float32),
        in_specs=[pl.BlockSpec(memory_space=pltpu.VMEM)] * 5,
        out_specs=pl.BlockSpec(memory_space=pltpu.VMEM),
        scratch_shapes=[
            pltpu.VMEM((N_DEV, BLK, D), jnp.float8_e4m3fn),
            pltpu.VMEM((N_DEV, BLK, D), jnp.float8_e4m3fn),
            pltpu.VMEM((SQ, HQ), jnp.bfloat16),
            pltpu.VMEM((N_DEV, BLK, HQ), jnp.bfloat16),
            pltpu.VMEM((SQ, D), jnp.bfloat16),
            pltpu.SemaphoreType.DMA((N_DEV,)),
            pltpu.SemaphoreType.DMA((N_DEV,)),
            pltpu.SemaphoreType.DMA((N_DEV,)),
            pltpu.SemaphoreType.DMA((N_DEV,)),
            pltpu.SemaphoreType.DMA((N_DEV,)),
            pltpu.SemaphoreType.DMA((N_DEV,)),
        ],
    )(x, Wq, K_ext, V_ext, Wo)
